# Initial kernel scaffold; baseline (speedup 1.0000x reference)
import jax
import jax.numpy as jnp
from jax.experimental import pallas as pl


def kernel(x, edge_index, edge_attr, W_enc, b_enc, eps, W1, b1, bn_gamma, bn_beta, W2, b2):
    raise NotImplementedError("write your pallas kernel here")


# speedup vs baseline: 1.7781x; 1.7781x over previous
import jax
import jax.numpy as jnp
from jax import lax
from jax.experimental import pallas as pl
from jax.experimental.pallas import tpu as pltpu
from jax.experimental.pallas import tpu_sc as plsc

N = 10000
E = 160000
D = 256
DH = 128
BN_EPS = 1e-5

NUM_SC = 2
NUM_TILES = 16
CHUNK = 32
EP = 163840
EDGES_PER_TILE = EP // NUM_TILES
CHUNKS_PER_TILE = EDGES_PER_TILE // CHUNK
SH_ROWS = 10240
ZROWS = SH_ROWS // NUM_TILES
OUT_ROWS = 640
OUT_ROWS_LAST = N - (NUM_TILES - 1) * OUT_ROWS


def _enc_body(attr_ref, w_ref, b_ref, el_ref, er_ref):
    e = jnp.dot(attr_ref[...], w_ref[...],
                preferred_element_type=jnp.float32) + b_ref[...]
    el_ref[...] = e[:, :DH]
    er_ref[...] = e[:, DH:]


def _encode(attr_pad, w_enc8, b_enc):
    BE = 2048
    grid = EP // BE
    return pl.pallas_call(
        _enc_body,
        grid=(grid,),
        in_specs=[
            pl.BlockSpec((BE, 8), lambda i: (i, 0)),
            pl.BlockSpec((8, D), lambda i: (0, 0)),
            pl.BlockSpec((1, D), lambda i: (0, 0)),
        ],
        out_specs=[
            pl.BlockSpec((BE, DH), lambda i: (i, 0)),
            pl.BlockSpec((BE, DH), lambda i: (i, 0)),
        ],
        out_shape=[
            jax.ShapeDtypeStruct((EP, DH), jnp.float32),
            jax.ShapeDtypeStruct((EP, DH), jnp.float32),
        ],
    )(attr_pad, w_enc8, b_enc)


def _sc_body(xl, xr, el, er, sd, zeros_hbm, out,
             sdv, dsc, xbuf, ebuf, mbuf, shared,
             sem_i, sem_x, sem_e, sem_s0, sem_s1):
    cid = lax.axis_index("c")
    tid = lax.axis_index("s")

    pltpu.sync_copy(zeros_hbm, shared.at[pl.ds(pl.multiple_of(tid * ZROWS, 8),
                                               ZROWS)])
    plsc.subcore_barrier()

    def run_half(x_h, e_h):
        def i_desc(k, slot):
            row = tid * CHUNKS_PER_TILE + k
            return pltpu.make_async_copy(sd.at[row], sdv.at[slot], sem_i)

        def g_desc(k, slot):
            return pltpu.make_async_copy(x_h.at[sdv.at[slot, 0]],
                                         xbuf.at[slot], sem_x)

        def e_desc(k, slot):
            base = pl.multiple_of(tid * EDGES_PER_TILE + k * CHUNK, 8)
            return pltpu.make_async_copy(e_h.at[pl.ds(base, CHUNK)],
                                         ebuf.at[slot], sem_e)

        def s_desc(k, slot):
            sem = sem_s0 if slot == 0 else sem_s1
            return pltpu.make_async_copy(mbuf.at[slot],
                                         shared.at[dsc.at[slot]], sem)

        i_desc(0, 0).start()
        i_desc(0, 0).wait()
        g_desc(0, 0).start()
        e_desc(0, 0).start()
        i_desc(1, 1).start()

        def body(i, carry):
            for sub in range(2):
                k = 2 * i + sub
                s = sub
                o = 1 - sub
                g_desc(k, s).wait()
                e_desc(k, s).wait()

                @pl.when(k >= 2)
                def _():
                    s_desc(k - 2, s).wait()

                @pl.when(k + 1 < CHUNKS_PER_TILE)
                def _():
                    i_desc(k + 1, o).wait()
                    g_desc(k + 1, o).start()
                    e_desc(k + 1, o).start()

                for j in range(CHUNK // 16):
                    sl = pl.ds(j * 16, 16)
                    dsc[s, sl] = sdv[s, 1, sl]

                @pl.when(k + 2 < CHUNKS_PER_TILE)
                def _():
                    i_desc(k + 2, s).start()

                def row(r, c2):
                    for j in range(DH // 16):
                        sl = pl.ds(j * 16, 16)
                        mbuf[s, r, sl] = jnp.maximum(
                            xbuf[s, r, sl] + ebuf[s, r, sl], 0.0)
                    return c2

                lax.fori_loop(0, CHUNK, row, 0)
                s_desc(k, s).start(add=True)
            return carry

        lax.fori_loop(0, CHUNKS_PER_TILE // 2, body, 0)
        s_desc(CHUNKS_PER_TILE - 2, 0).wait()
        s_desc(CHUNKS_PER_TILE - 1, 1).wait()

    @pl.when(cid == 0)
    def _():
        run_half(xl, el)

    @pl.when(cid == 1)
    def _():
        run_half(xr, er)

    plsc.subcore_barrier()

    r0 = pl.multiple_of(tid * OUT_ROWS, 8)
    col = pl.multiple_of(cid * DH, DH)

    @pl.when(tid < NUM_TILES - 1)
    def _():
        pltpu.sync_copy(shared.at[pl.ds(r0, OUT_ROWS)],
                        out.at[pl.ds(r0, OUT_ROWS), pl.ds(col, DH)])

    @pl.when(tid == NUM_TILES - 1)
    def _():
        pltpu.sync_copy(shared.at[pl.ds(r0, OUT_ROWS_LAST)],
                        out.at[pl.ds(r0, OUT_ROWS_LAST), pl.ds(col, DH)])


_sc_aggregate = pl.kernel(
    _sc_body,
    out_type=jax.ShapeDtypeStruct((N, D), jnp.float32),
    mesh=plsc.VectorSubcoreMesh(core_axis_name="c", subcore_axis_name="s"),
    scratch_types=[
        pltpu.VMEM((2, 2, CHUNK), jnp.int32),
        pltpu.VMEM((2, CHUNK), jnp.int32),
        pltpu.VMEM((2, CHUNK, DH), jnp.float32),
        pltpu.VMEM((2, CHUNK, DH), jnp.float32),
        pltpu.VMEM((2, CHUNK, DH), jnp.float32),
        pltpu.VMEM_SHARED((SH_ROWS, DH), jnp.float32),
        pltpu.SemaphoreType.DMA,
        pltpu.SemaphoreType.DMA,
        pltpu.SemaphoreType.DMA,
        pltpu.SemaphoreType.DMA,
        pltpu.SemaphoreType.DMA,
    ],
)


_BR = 400
_NB = N // _BR


def _mlp_body(scale_ref, x_ref, aggr_ref, w1_ref, b1_ref, gamma_ref, beta_ref,
              w2_ref, b2_ref, out_ref, h1_scr, sum_scr, sq_scr):
    i = pl.program_id(0)

    @pl.when(i == 0)
    def _():
        sum_scr[...] = jnp.zeros_like(sum_scr)
        sq_scr[...] = jnp.zeros_like(sq_scr)

    @pl.when(i < _NB)
    def _():
        h = scale_ref[0, 0] * x_ref[...] + aggr_ref[...]
        h1 = (jnp.dot(h, w1_ref[...], preferred_element_type=jnp.float32)
              + b1_ref[...])
        h1_scr[pl.ds(i * _BR, _BR), :] = h1
        sum_scr[...] += jnp.sum(h1, axis=0, keepdims=True)
        sq_scr[...] += jnp.sum(h1 * h1, axis=0, keepdims=True)

    @pl.when(i >= _NB)
    def _():
        mu = sum_scr[...] / N
        var = sq_scr[...] / N - mu * mu
        a = gamma_ref[...] * lax.rsqrt(var + BN_EPS)
        c = beta_ref[...] - mu * a
        h1 = h1_scr[pl.ds((i - _NB) * _BR, _BR), :]
        y = jnp.maximum(h1 * a + c, 0.0)
        out_ref[...] = jnp.dot(y, w2_ref[...],
                               preferred_element_type=jnp.float32) + b2_ref[...]


def _mlp(scale, x, aggr, w1, b1, gamma, beta, w2, b2):
    def blk(i):
        return (lax.rem(i, _NB), 0)

    def blk_in(i):
        return (jnp.where(i < _NB, i, 0), 0)

    return pl.pallas_call(
        _mlp_body,
        grid=(2 * _NB,),
        in_specs=[
            pl.BlockSpec(memory_space=pltpu.SMEM),
            pl.BlockSpec((_BR, D), blk_in),
            pl.BlockSpec((_BR, D), blk_in),
            pl.BlockSpec((D, 2 * D), lambda i: (0, 0)),
            pl.BlockSpec((1, 2 * D), lambda i: (0, 0)),
            pl.BlockSpec((1, 2 * D), lambda i: (0, 0)),
            pl.BlockSpec((1, 2 * D), lambda i: (0, 0)),
            pl.BlockSpec((2 * D, D), lambda i: (0, 0)),
            pl.BlockSpec((1, D), lambda i: (0, 0)),
        ],
        out_specs=pl.BlockSpec((_BR, D), blk),
        out_shape=jax.ShapeDtypeStruct((N, D), jnp.float32),
        scratch_shapes=[
            pltpu.VMEM((N, 2 * D), jnp.float32),
            pltpu.VMEM((1, 2 * D), jnp.float32),
            pltpu.VMEM((1, 2 * D), jnp.float32),
        ],
    )(scale, x, aggr, w1, b1, gamma, beta, w2, b2)


def kernel(x, edge_index, edge_attr, W_enc, b_enc, eps, W1, b1, bn_gamma,
           bn_beta, W2, b2):
    src = edge_index[0].astype(jnp.int32)
    dst = edge_index[1].astype(jnp.int32)
    src_p = jnp.pad(src, (0, EP - E)).reshape(EP // CHUNK, CHUNK)
    dst_p = jnp.pad(dst, (0, EP - E), constant_values=N).reshape(
        EP // CHUNK, CHUNK)
    sd = jnp.stack([src_p, dst_p], axis=1)
    attr_p = jnp.pad(edge_attr, ((0, EP - E), (0, 1)))
    w_enc8 = jnp.pad(W_enc, ((0, 1), (0, 0)))
    xl = x[:, :DH]
    xr = x[:, DH:]
    zeros_blk = jnp.zeros((ZROWS, DH), jnp.float32)
    scale = jnp.reshape(1.0 + eps, (1, 1))

    el, er = _encode(attr_p, w_enc8, jnp.reshape(b_enc, (1, D)))

    aggr = _sc_aggregate(xl, xr, el, er, sd, zeros_blk)

    out = _mlp(scale, x, aggr, W1, jnp.reshape(b1, (1, 2 * D)),
               jnp.reshape(bn_gamma, (1, 2 * D)),
               jnp.reshape(bn_beta, (1, 2 * D)), W2, jnp.reshape(b2, (1, D)))
    return out
